# baseline (speedup 1.0000x reference)
import jax
import jax.numpy as jnp
from jax import lax
from jax.experimental import pallas as pl

NUM_SEQ = 8192
NUM_LABEL = 29
BLANK = 0


def _ctc_body(em_ref, oind_ref, ovalid_ref, obest_ref):
    em = em_ref[...]
    ind = jnp.argmax(em, axis=1).reshape(NUM_SEQ, 1).astype(jnp.int32)
    best = jnp.max(em, axis=1).reshape(NUM_SEQ, 1)
    prev = pltpu_roll(ind)
    row = lax.broadcasted_iota(jnp.int32, (NUM_SEQ, 1), 0)
    change = (ind != prev) | (row == 0)
    valid = change & (ind != BLANK)
    oind_ref[...] = jnp.where(valid, ind, -1)
    ovalid_ref[...] = valid.astype(jnp.int32)
    obest_ref[...] = best


def pltpu_roll(x):
    return jnp.roll(x, 1, axis=0)


@jax.jit
def kernel(emission):
    oind, ovalid, obest = pl.pallas_call(
        _ctc_body,
        out_shape=[
            jax.ShapeDtypeStruct((NUM_SEQ, 1), jnp.int32),
            jax.ShapeDtypeStruct((NUM_SEQ, 1), jnp.int32),
            jax.ShapeDtypeStruct((NUM_SEQ, 1), jnp.float32),
        ],
    )(emission)
    return (
        oind.reshape(NUM_SEQ),
        ovalid.reshape(NUM_SEQ) != 0,
        obest.reshape(NUM_SEQ),
    )


# speedup vs baseline: 1.7716x; 1.7716x over previous
import jax
import jax.numpy as jnp
from jax import lax
from jax.experimental import pallas as pl
from jax.experimental.pallas import tpu as pltpu

NUM_SEQ = 8192
NUM_LABEL = 29
BLANK = 0

BLK = 1024
GRID = NUM_SEQ // BLK
SUB = BLK // 128


def _ctc_body(em_ref, oind_ref, ovalid_ref, obest_ref, carry_ref):
    b = pl.program_id(0)
    em = em_ref[...]
    emt = em.T
    best = jnp.max(emt, axis=0).reshape(SUB, 128)
    ind = jnp.argmax(emt, axis=0).astype(jnp.int32).reshape(SUB, 128)

    def shift_flat(x):
        lane = lax.broadcasted_iota(jnp.int32, (SUB, 128), 1)
        r1 = pltpu.roll(x, 1, 1)
        r2 = pltpu.roll(pltpu.roll(x, 1, 0), 1, 1)
        return jnp.where(lane == 0, r2, r1)

    row = lax.broadcasted_iota(jnp.int32, (SUB, 128), 0)
    lane = lax.broadcasted_iota(jnp.int32, (SUB, 128), 1)
    at00 = (row == 0) & (lane == 0)
    prev = jnp.where(at00, shift_flat(carry_ref[...]), shift_flat(ind))
    change = (ind != prev) | (at00 & (b == 0))
    valid = change & (ind != BLANK)
    oind_ref[...] = jnp.where(valid, ind, -1)
    ovalid_ref[...] = valid.astype(jnp.int32)
    obest_ref[...] = best
    carry_ref[...] = ind


@jax.jit
def kernel(emission):
    oind, ovalid, obest = pl.pallas_call(
        _ctc_body,
        grid=(GRID,),
        in_specs=[pl.BlockSpec((BLK, NUM_LABEL), lambda i: (i, 0))],
        out_specs=[
            pl.BlockSpec((SUB, 128), lambda i: (i, 0)),
            pl.BlockSpec((SUB, 128), lambda i: (i, 0)),
            pl.BlockSpec((SUB, 128), lambda i: (i, 0)),
        ],
        out_shape=[
            jax.ShapeDtypeStruct((GRID * SUB, 128), jnp.int32),
            jax.ShapeDtypeStruct((GRID * SUB, 128), jnp.int32),
            jax.ShapeDtypeStruct((GRID * SUB, 128), jnp.float32),
        ],
        scratch_shapes=[pltpu.VMEM((SUB, 128), jnp.int32)],
    )(emission)
    return (
        oind.reshape(NUM_SEQ),
        ovalid.reshape(NUM_SEQ) != 0,
        obest.reshape(NUM_SEQ),
    )
